# initial kernel scaffold (unmeasured)
import jax
import jax.numpy as jnp
from jax import lax
from jax.experimental import pallas as pl
from jax.experimental.pallas import tpu as pltpu

N_DEV = 4


def kernel(x, Wq, K_ext, V_ext, Wo):
    B, Sq, D = x.shape
    _, Skv_loc, Hq, Dh = K_ext.shape
    H_loc = Wq.shape[1] // Dh
    HD_loc = H_loc * Dh

    x2 = x.reshape(B * Sq, D)
    k2 = K_ext.reshape(B, Skv_loc, Hq * Dh)
    v2 = V_ext.reshape(B, Skv_loc, Hq * Dh)

    def body(x_ref, wq_ref, k_ref, v_ref, wo_ref, out_ref,
             k_full, v_full, acc,
             send_sems, recv_sems, ar_send_sems, ar_recv_sems, local_sems):
        me = lax.axis_index("i")

        barrier_sem = pltpu.get_barrier_semaphore()
        for d in range(1, N_DEV):
            peer = (me + d) % N_DEV
            pl.semaphore_signal(barrier_sem, inc=1, device_id=(peer,),
                                device_id_type=pl.DeviceIdType.MESH)
        pl.semaphore_wait(barrier_sem, N_DEV - 1)

        a2a = []
        for d in range(1, N_DEV):
            peer = (me + d) % N_DEV
            for t, (src, dst) in enumerate(((k_ref, k_full), (v_ref, v_full))):
                rdma = pltpu.make_async_remote_copy(
                    src_ref=src.at[:, :, pl.ds(peer * HD_loc, HD_loc)],
                    dst_ref=dst.at[me],
                    send_sem=send_sems.at[t, d - 1],
                    recv_sem=recv_sems.at[t, d - 1],
                    device_id=(peer,),
                    device_id_type=pl.DeviceIdType.MESH,
                )
                rdma.start()
                a2a.append(rdma)

        local_copies = []
        for t, (src, dst) in enumerate(((k_ref, k_full), (v_ref, v_full))):
            cp = pltpu.make_async_copy(
                src.at[:, :, pl.ds(me * HD_loc, HD_loc)],
                dst.at[me],
                local_sems.at[t],
            )
            cp.start()
            local_copies.append(cp)

        q2d = jnp.dot(x_ref[...], wq_ref[...],
                      preferred_element_type=jnp.float32)

        Skv = N_DEV * Skv_loc
        qi = lax.broadcasted_iota(jnp.int32, (Sq, Skv), 0)
        ki = lax.broadcasted_iota(jnp.int32, (Sq, Skv), 1)
        mask = (jnp.abs(qi - ki) <= 128) | (ki < 32) | (qi < 32)
        neg = jnp.float32(-1e9)

        for cp in local_copies:
            cp.wait()
        for rdma in a2a:
            rdma.wait()

        kf = k_full[...]
        vf = v_full[...]

        ctx_rows = []
        for b in range(B):
            q_b = q2d[b * Sq:(b + 1) * Sq, :]
            ctx_cols = []
            for h in range(H_loc):
                q_bh = q_b[:, h * Dh:(h + 1) * Dh]
                k_cat = jnp.concatenate(
                    [kf[s, b, :, h * Dh:(h + 1) * Dh] for s in range(N_DEV)],
                    axis=0)
                v_cat = jnp.concatenate(
                    [vf[s, b, :, h * Dh:(h + 1) * Dh] for s in range(N_DEV)],
                    axis=0)
                scores = lax.dot_general(
                    q_bh, k_cat, (((1,), (1,)), ((), ())),
                    preferred_element_type=jnp.float32) * 0.125
                scores = jnp.where(mask, scores, neg)
                m = jnp.max(scores, axis=-1, keepdims=True)
                w = jnp.exp(scores - m)
                w = w / jnp.sum(w, axis=-1, keepdims=True)
                ctx_cols.append(jnp.dot(w, v_cat,
                                        preferred_element_type=jnp.float32))
            ctx_rows.append(jnp.concatenate(ctx_cols, axis=1))
        ctx2d = jnp.concatenate(ctx_rows, axis=0)

        acc[0] = jnp.dot(ctx2d, wo_ref[...],
                         preferred_element_type=jnp.float32)

        ar = []
        for d in range(1, N_DEV):
            peer = (me + d) % N_DEV
            rdma = pltpu.make_async_remote_copy(
                src_ref=acc.at[0],
                dst_ref=acc.at[d],
                send_sem=ar_send_sems.at[d - 1],
                recv_sem=ar_recv_sems.at[d - 1],
                device_id=(peer,),
                device_id_type=pl.DeviceIdType.MESH,
            )
            rdma.start()
            ar.append(rdma)
        for rdma in ar:
            rdma.wait()

        out_ref[...] = jnp.sum(acc[...], axis=0)

    out2 = pl.pallas_call(
        body,
        out_shape=jax.ShapeDtypeStruct((B * Sq, D), jnp.float32),
        in_specs=[pl.BlockSpec(memory_space=pltpu.VMEM)] * 5,
        out_specs=pl.BlockSpec(memory_space=pltpu.VMEM),
        scratch_shapes=[
            pltpu.VMEM((N_DEV, B, Skv_loc, HD_loc), jnp.float32),
            pltpu.VMEM((N_DEV, B, Skv_loc, HD_loc), jnp.float32),
            pltpu.VMEM((N_DEV, B * Sq, D), jnp.float32),
            pltpu.SemaphoreType.DMA((2, N_DEV - 1)),
            pltpu.SemaphoreType.DMA((2, N_DEV - 1)),
            pltpu.SemaphoreType.DMA((N_DEV - 1,)),
            pltpu.SemaphoreType.DMA((N_DEV - 1,)),
            pltpu.SemaphoreType.DMA((2,)),
        ],
        compiler_params=pltpu.CompilerParams(collective_id=0),
    )(x2, wq_ref_arg := Wq, k2, v2, Wo)

    return out2.reshape(B, Sq, D)


# baseline (device time: 37334 ns/iter reference)
import jax
import jax.numpy as jnp
from jax import lax
from jax.experimental import pallas as pl
from jax.experimental.pallas import tpu as pltpu

N_DEV = 4


def kernel(x, Wq, K_ext, V_ext, Wo):
    B, Sq, D = x.shape
    _, Skv_loc, Hq, Dh = K_ext.shape
    H_loc = Wq.shape[1] // Dh
    HD_loc = H_loc * Dh

    x2 = x.reshape(B * Sq, D)
    k2 = K_ext.reshape(B, Skv_loc, Hq * Dh)
    v2 = V_ext.reshape(B, Skv_loc, Hq * Dh)

    def body(x_ref, wq_ref, k_ref, v_ref, wo_ref, out_ref,
             k_full, v_full, acc,
             send_sems, recv_sems, ar_send_sems, ar_recv_sems, local_sems):
        me = lax.axis_index("i")

        barrier_sem = pltpu.get_barrier_semaphore()
        for d in range(1, N_DEV):
            peer = (me + d) % N_DEV
            pl.semaphore_signal(barrier_sem, inc=1, device_id=(peer,),
                                device_id_type=pl.DeviceIdType.MESH)
        pl.semaphore_wait(barrier_sem, N_DEV - 1)

        a2a = []
        for d in range(1, N_DEV):
            peer = (me + d) % N_DEV
            for t, (src, dst) in enumerate(((k_ref, k_full), (v_ref, v_full))):
                rdma = pltpu.make_async_remote_copy(
                    src_ref=src.at[:, :, pl.ds(peer * HD_loc, HD_loc)],
                    dst_ref=dst.at[me],
                    send_sem=send_sems.at[t, d - 1],
                    recv_sem=recv_sems.at[t, d - 1],
                    device_id=(peer,),
                    device_id_type=pl.DeviceIdType.MESH,
                )
                rdma.start()
                a2a.append(rdma)

        local_copies = []
        for t, (src, dst) in enumerate(((k_ref, k_full), (v_ref, v_full))):
            cp = pltpu.make_async_copy(
                src.at[:, :, pl.ds(me * HD_loc, HD_loc)],
                dst.at[me],
                local_sems.at[t],
            )
            cp.start()
            local_copies.append(cp)

        q2d = jnp.dot(x_ref[...], wq_ref[...],
                      preferred_element_type=jnp.float32)

        Skv = N_DEV * Skv_loc
        qi = lax.broadcasted_iota(jnp.int32, (Sq, Skv), 0)
        ki = lax.broadcasted_iota(jnp.int32, (Sq, Skv), 1)
        mask = (jnp.abs(qi - ki) <= 128) | (ki < 32) | (qi < 32)
        neg = jnp.float32(-1e9)

        for cp in local_copies:
            cp.wait()
        for rdma in a2a:
            rdma.wait()

        kf = k_full[...]
        vf = v_full[...]

        ctx_rows = []
        for b in range(B):
            q_b = q2d[b * Sq:(b + 1) * Sq, :]
            ctx_cols = []
            for h in range(H_loc):
                q_bh = q_b[:, h * Dh:(h + 1) * Dh]
                k_cat = jnp.concatenate(
                    [kf[s, b, :, h * Dh:(h + 1) * Dh] for s in range(N_DEV)],
                    axis=0)
                v_cat = jnp.concatenate(
                    [vf[s, b, :, h * Dh:(h + 1) * Dh] for s in range(N_DEV)],
                    axis=0)
                scores = lax.dot_general(
                    q_bh, k_cat, (((1,), (1,)), ((), ())),
                    preferred_element_type=jnp.float32) * 0.125
                scores = jnp.where(mask, scores, neg)
                m = jnp.max(scores, axis=-1, keepdims=True)
                w = jnp.exp(scores - m)
                w = w / jnp.sum(w, axis=-1, keepdims=True)
                ctx_cols.append(jnp.dot(w, v_cat,
                                        preferred_element_type=jnp.float32))
            ctx_rows.append(jnp.concatenate(ctx_cols, axis=1))
        ctx2d = jnp.concatenate(ctx_rows, axis=0)

        acc[0] = jnp.dot(ctx2d, wo_ref[...],
                         preferred_element_type=jnp.float32)

        ar = []
        for d in range(1, N_DEV):
            peer = (me + d) % N_DEV
            rdma = pltpu.make_async_remote_copy(
                src_ref=acc.at[0],
                dst_ref=acc.at[d],
                send_sem=ar_send_sems.at[d - 1],
                recv_sem=ar_recv_sems.at[d - 1],
                device_id=(peer,),
                device_id_type=pl.DeviceIdType.MESH,
            )
            rdma.start()
            ar.append(rdma)
        for rdma in ar:
            rdma.wait()

        out_ref[...] = jnp.sum(acc[...], axis=0)

    out2 = pl.pallas_call(
        body,
        out_shape=jax.ShapeDtypeStruct((B * Sq, D), jnp.float32),
        in_specs=[pl.BlockSpec(memory_space=pltpu.VMEM)] * 5,
        out_specs=pl.BlockSpec(memory_space=pltpu.VMEM),
        scratch_shapes=[
            pltpu.VMEM((N_DEV, B, Skv_loc, HD_loc), jnp.float32),
            pltpu.VMEM((N_DEV, B, Skv_loc, HD_loc), jnp.float32),
            pltpu.VMEM((N_DEV, B * Sq, D), jnp.float32),
            pltpu.SemaphoreType.DMA((2, N_DEV - 1)),
            pltpu.SemaphoreType.DMA((2, N_DEV - 1)),
            pltpu.SemaphoreType.DMA((N_DEV - 1,)),
            pltpu.SemaphoreType.DMA((N_DEV - 1,)),
            pltpu.SemaphoreType.DMA((2,)),
        ],
        compiler_params=pltpu.CompilerParams(collective_id=0),
    )(x2, Wq, k2, v2, Wo)

    return out2.reshape(B, Sq, D)


# device time: 26165 ns/iter; 1.4269x vs baseline; 1.4269x over previous
import jax
import jax.numpy as jnp
from jax import lax
from jax.experimental import pallas as pl
from jax.experimental.pallas import tpu as pltpu

N_DEV = 4


def kernel(x, Wq, K_ext, V_ext, Wo):
    B, Sq, D = x.shape
    _, Skv_loc, Hq, Dh = K_ext.shape
    H_loc = Wq.shape[1] // Dh
    HD_loc = H_loc * Dh

    x2 = x.reshape(B * Sq, D)
    k2 = K_ext.reshape(B, Skv_loc, Hq * Dh)
    v2 = V_ext.reshape(B, Skv_loc, Hq * Dh)

    def body(x_ref, wq_ref, k_ref, v_ref, wo_ref, out_ref,
             k_stage, v_stage, k_full, v_full, acc,
             send_sems, recv_sems, ar_send_sems, ar_recv_sems, local_sems):
        me = lax.axis_index("i")

        barrier_sem = pltpu.get_barrier_semaphore()
        for d in range(1, N_DEV):
            peer = (me + d) % N_DEV
            pl.semaphore_signal(barrier_sem, inc=1, device_id=(peer,),
                                device_id_type=pl.DeviceIdType.MESH)
        pl.semaphore_wait(barrier_sem, N_DEV - 1)

        k_stage[...] = k_ref[...].astype(jnp.bfloat16)
        v_stage[...] = v_ref[...].astype(jnp.bfloat16)

        a2a = []
        for d in range(1, N_DEV):
            peer = (me + d) % N_DEV
            for t, (src, dst) in enumerate(((k_stage, k_full),
                                            (v_stage, v_full))):
                rdma = pltpu.make_async_remote_copy(
                    src_ref=src.at[:, :, pl.ds(peer * HD_loc, HD_loc)],
                    dst_ref=dst.at[me],
                    send_sem=send_sems.at[t, d - 1],
                    recv_sem=recv_sems.at[t, d - 1],
                    device_id=(peer,),
                    device_id_type=pl.DeviceIdType.MESH,
                )
                rdma.start()
                a2a.append(rdma)

        local_copies = []
        for t, (src, dst) in enumerate(((k_stage, k_full),
                                        (v_stage, v_full))):
            cp = pltpu.make_async_copy(
                src.at[:, :, pl.ds(me * HD_loc, HD_loc)],
                dst.at[me],
                local_sems.at[t],
            )
            cp.start()
            local_copies.append(cp)

        q2d = jnp.dot(x_ref[...], wq_ref[...],
                      preferred_element_type=jnp.float32)

        Skv = N_DEV * Skv_loc
        qi = lax.broadcasted_iota(jnp.int32, (Sq, Skv), 0)
        ki = lax.broadcasted_iota(jnp.int32, (Sq, Skv), 1)
        mask = (jnp.abs(qi - ki) <= 128) | (ki < 32) | (qi < 32)
        neg = jnp.float32(-1e9)

        for cp in local_copies:
            cp.wait()
        for rdma in a2a:
            rdma.wait()

        kf = k_full[...]
        vf = v_full[...]

        ctx_rows = []
        for b in range(B):
            q_b = q2d[b * Sq:(b + 1) * Sq, :]
            ctx_cols = []
            for h in range(H_loc):
                q_bh = q_b[:, h * Dh:(h + 1) * Dh]
                k_cat = jnp.concatenate(
                    [kf[s, b, :, h * Dh:(h + 1) * Dh] for s in range(N_DEV)],
                    axis=0).astype(jnp.float32)
                v_cat = jnp.concatenate(
                    [vf[s, b, :, h * Dh:(h + 1) * Dh] for s in range(N_DEV)],
                    axis=0).astype(jnp.float32)
                scores = lax.dot_general(
                    q_bh, k_cat, (((1,), (1,)), ((), ())),
                    preferred_element_type=jnp.float32) * 0.125
                scores = jnp.where(mask, scores, neg)
                m = jnp.max(scores, axis=-1, keepdims=True)
                w = jnp.exp(scores - m)
                w = w / jnp.sum(w, axis=-1, keepdims=True)
                ctx_cols.append(jnp.dot(w, v_cat,
                                        preferred_element_type=jnp.float32))
            ctx_rows.append(jnp.concatenate(ctx_cols, axis=1))
        ctx2d = jnp.concatenate(ctx_rows, axis=0)

        partial = jnp.dot(ctx2d, wo_ref[...],
                          preferred_element_type=jnp.float32)
        acc[0] = partial.astype(jnp.bfloat16)

        ar = []
        for d in range(1, N_DEV):
            peer = (me + d) % N_DEV
            rdma = pltpu.make_async_remote_copy(
                src_ref=acc.at[0],
                dst_ref=acc.at[d],
                send_sem=ar_send_sems.at[d - 1],
                recv_sem=ar_recv_sems.at[d - 1],
                device_id=(peer,),
                device_id_type=pl.DeviceIdType.MESH,
            )
            rdma.start()
            ar.append(rdma)
        for rdma in ar:
            rdma.wait()

        others = acc[...].astype(jnp.float32)
        out_ref[...] = partial + others[1] + others[2] + others[3]

    out2 = pl.pallas_call(
        body,
        out_shape=jax.ShapeDtypeStruct((B * Sq, D), jnp.float32),
        in_specs=[pl.BlockSpec(memory_space=pltpu.VMEM)] * 5,
        out_specs=pl.BlockSpec(memory_space=pltpu.VMEM),
        scratch_shapes=[
            pltpu.VMEM((B, Skv_loc, Hq * Dh), jnp.bfloat16),
            pltpu.VMEM((B, Skv_loc, Hq * Dh), jnp.bfloat16),
            pltpu.VMEM((N_DEV, B, Skv_loc, HD_loc), jnp.bfloat16),
            pltpu.VMEM((N_DEV, B, Skv_loc, HD_loc), jnp.bfloat16),
            pltpu.VMEM((N_DEV, B * Sq, D), jnp.bfloat16),
            pltpu.SemaphoreType.DMA((2, N_DEV - 1)),
            pltpu.SemaphoreType.DMA((2, N_DEV - 1)),
            pltpu.SemaphoreType.DMA((N_DEV - 1,)),
            pltpu.SemaphoreType.DMA((N_DEV - 1,)),
            pltpu.SemaphoreType.DMA((2,)),
        ],
        compiler_params=pltpu.CompilerParams(collective_id=0),
    )(x2, Wq, k2, v2, Wo)

    return out2.reshape(B, Sq, D)


# device time: 26013 ns/iter; 1.4352x vs baseline; 1.0058x over previous
import jax
import jax.numpy as jnp
from jax import lax
from jax.experimental import pallas as pl
from jax.experimental.pallas import tpu as pltpu

N_DEV = 4


def kernel(x, Wq, K_ext, V_ext, Wo):
    B, Sq, D = x.shape
    _, Skv_loc, Hq, Dh = K_ext.shape
    H_loc = Wq.shape[1] // Dh
    HD_loc = H_loc * Dh

    x2 = x.reshape(B * Sq, D)
    k2 = K_ext.reshape(B, Skv_loc, Hq * Dh)
    v2 = V_ext.reshape(B, Skv_loc, Hq * Dh)

    def body(x_ref, wq_ref, k_ref, v_ref, wo_ref, out_ref,
             k_stage, v_stage, k_full, v_full, acc,
             send_sems, recv_sems, ar_send_sems, ar_recv_sems, local_sems):
        me = lax.axis_index("i")

        barrier_sem = pltpu.get_barrier_semaphore()
        for d in range(1, N_DEV):
            peer = (me + d) % N_DEV
            pl.semaphore_signal(barrier_sem, inc=1, device_id=(peer,),
                                device_id_type=pl.DeviceIdType.MESH)
        pl.semaphore_wait(barrier_sem, N_DEV - 1)

        k_stage[...] = k_ref[...].astype(jnp.bfloat16)
        v_stage[...] = v_ref[...].astype(jnp.bfloat16)

        a2a = []
        for d in (2, 1, 3):
            peer = (me + d) % N_DEV
            for t, (src, dst) in enumerate(((k_stage, k_full),
                                            (v_stage, v_full))):
                rdma = pltpu.make_async_remote_copy(
                    src_ref=src.at[:, :, pl.ds(peer * HD_loc, HD_loc)],
                    dst_ref=dst.at[me],
                    send_sem=send_sems.at[t, d - 1],
                    recv_sem=recv_sems.at[t, d - 1],
                    device_id=(peer,),
                    device_id_type=pl.DeviceIdType.MESH,
                )
                rdma.start()
                a2a.append(rdma)

        local_copies = []
        for t, (src, dst) in enumerate(((k_stage, k_full),
                                        (v_stage, v_full))):
            cp = pltpu.make_async_copy(
                src.at[:, :, pl.ds(me * HD_loc, HD_loc)],
                dst.at[me],
                local_sems.at[t],
            )
            cp.start()
            local_copies.append(cp)

        q2d = jnp.dot(x_ref[...], wq_ref[...],
                      preferred_element_type=jnp.float32)

        Skv = N_DEV * Skv_loc
        qi = lax.broadcasted_iota(jnp.int32, (Sq, Skv), 0)
        ki = lax.broadcasted_iota(jnp.int32, (Sq, Skv), 1)
        mask = (jnp.abs(qi - ki) <= 128) | (ki < 32) | (qi < 32)
        neg = jnp.float32(-1e9)

        for cp in local_copies:
            cp.wait()
        for rdma in a2a:
            rdma.wait()

        kf = k_full[...]
        vf = v_full[...]

        ar = []
        partials = []
        for b in range(B):
            q_b = q2d[b * Sq:(b + 1) * Sq, :]
            ctx_cols = []
            for h in range(H_loc):
                q_bh = q_b[:, h * Dh:(h + 1) * Dh]
                k_cat = jnp.concatenate(
                    [kf[s, b, :, h * Dh:(h + 1) * Dh] for s in range(N_DEV)],
                    axis=0).astype(jnp.float32)
                v_cat = jnp.concatenate(
                    [vf[s, b, :, h * Dh:(h + 1) * Dh] for s in range(N_DEV)],
                    axis=0).astype(jnp.float32)
                scores = lax.dot_general(
                    q_bh, k_cat, (((1,), (1,)), ((), ())),
                    preferred_element_type=jnp.float32) * 0.125
                scores = jnp.where(mask, scores, neg)
                m = jnp.max(scores, axis=-1, keepdims=True)
                w = jnp.exp(scores - m)
                w = w / jnp.sum(w, axis=-1, keepdims=True)
                ctx_cols.append(jnp.dot(w, v_cat,
                                        preferred_element_type=jnp.float32))
            ctx_b = jnp.concatenate(ctx_cols, axis=1)
            partial_b = jnp.dot(ctx_b, wo_ref[...],
                                preferred_element_type=jnp.float32)
            partials.append(partial_b)
            acc[0, pl.ds(b * Sq, Sq)] = partial_b.astype(jnp.bfloat16)
            for d in (2, 1, 3):
                peer = (me + d) % N_DEV
                rdma = pltpu.make_async_remote_copy(
                    src_ref=acc.at[0, pl.ds(b * Sq, Sq)],
                    dst_ref=acc.at[d, pl.ds(b * Sq, Sq)],
                    send_sem=ar_send_sems.at[b, d - 1],
                    recv_sem=ar_recv_sems.at[b, d - 1],
                    device_id=(peer,),
                    device_id_type=pl.DeviceIdType.MESH,
                )
                rdma.start()
                ar.append(rdma)
        for rdma in ar:
            rdma.wait()

        partial = jnp.concatenate(partials, axis=0)
        others = acc[...].astype(jnp.float32)
        out_ref[...] = partial + others[1] + others[2] + others[3]

    out2 = pl.pallas_call(
        body,
        out_shape=jax.ShapeDtypeStruct((B * Sq, D), jnp.float32),
        in_specs=[pl.BlockSpec(memory_space=pltpu.VMEM)] * 5,
        out_specs=pl.BlockSpec(memory_space=pltpu.VMEM),
        scratch_shapes=[
            pltpu.VMEM((B, Skv_loc, Hq * Dh), jnp.bfloat16),
            pltpu.VMEM((B, Skv_loc, Hq * Dh), jnp.bfloat16),
            pltpu.VMEM((N_DEV, B, Skv_loc, HD_loc), jnp.bfloat16),
            pltpu.VMEM((N_DEV, B, Skv_loc, HD_loc), jnp.bfloat16),
            pltpu.VMEM((N_DEV, B * Sq, D), jnp.bfloat16),
            pltpu.SemaphoreType.DMA((2, N_DEV - 1)),
            pltpu.SemaphoreType.DMA((2, N_DEV - 1)),
            pltpu.SemaphoreType.DMA((B, N_DEV - 1)),
            pltpu.SemaphoreType.DMA((B, N_DEV - 1)),
            pltpu.SemaphoreType.DMA((2,)),
        ],
        compiler_params=pltpu.CompilerParams(collective_id=0),
    )(x2, Wq, k2, v2, Wo)

    return out2.reshape(B, Sq, D)
